# baseline (speedup 1.0000x reference)
import functools

import jax
import jax.numpy as jnp
from jax import lax
from jax.experimental import pallas as pl
from jax.experimental.pallas import tpu as pltpu
from jax.experimental.pallas import tpu_sc as plsc

MIN_NORM = 1e-7
BALL_EPS = 1e-5

NC = 2
NS = 16
NW = NC * NS

CHUNK = 50
NBUF = 4
IDX_BLOCK = 8
CNT_PIECE = 640


def _transform_body(x_ref, w_ref, b_ref, h_ref):
    w = w_ref[...]
    b = b_ref[...]
    wn = jnp.maximum(jnp.sqrt(jnp.sum(w * w, axis=-1, keepdims=True)), MIN_NORM)
    w_h = jnp.tanh(wn) * w / wn
    bn = jnp.maximum(jnp.sqrt(jnp.sum(b * b, axis=-1, keepdims=True)), MIN_NORM)
    b_h = jnp.tanh(bn) * b / bn
    x = x_ref[...]
    xn = jnp.maximum(jnp.sqrt(jnp.sum(x * x, axis=-1, keepdims=True)), MIN_NORM)
    mx = jnp.dot(x, w_h, preferred_element_type=jnp.float32)
    mxn = jnp.maximum(jnp.sqrt(jnp.sum(mx * mx, axis=-1, keepdims=True)), MIN_NORM)
    xc = jnp.clip(xn, -1.0 + 1e-7, 1.0 - 1e-7)
    art = 0.5 * jnp.log((1.0 + xc) / (1.0 - xc))
    res = jnp.tanh(mxn / xn * art) * mx / mxn
    nz = jnp.max(jnp.abs(mx), axis=-1, keepdims=True) > 0.0
    h = jnp.where(nz, res, 0.0)
    x2 = jnp.sum(h * h, axis=-1, keepdims=True)
    y2 = jnp.sum(b_h * b_h, axis=-1, keepdims=True)
    xy = jnp.sum(h * b_h, axis=-1, keepdims=True)
    num = (1.0 + 2.0 * xy + y2) * h + (1.0 - x2) * b_h
    den = 1.0 + 2.0 * xy + x2 * y2
    h_ref[...] = num / jnp.maximum(den, 1e-15)


def _combine_body(sums_ref, cnts_ref, out_ref):
    s = sums_ref[0] + sums_ref[1]
    c = cnts_ref[0] + cnts_ref[1]
    agg = s / jnp.maximum(c, 1.0)[:, None]
    norm = jnp.maximum(jnp.sqrt(jnp.sum(agg * agg, axis=-1, keepdims=True)), MIN_NORM)
    maxnorm = 1.0 - BALL_EPS
    out_ref[...] = jnp.where(norm > maxnorm, agg / norm * maxnorm, agg)


def _make_sc_agg(n_nodes, n_edges, d):
    edges_per_w = n_edges // NW
    n_chunks = edges_per_w // CHUNK
    rows_per_tile = (n_nodes // NS) // 8 * 8
    rem_row0 = NS * rows_per_tile
    rem_rows = n_nodes - rem_row0
    mesh = plsc.VectorSubcoreMesh(core_axis_name="c", subcore_axis_name="s")

    @functools.partial(
        pl.kernel,
        out_type=[
            jax.ShapeDtypeStruct((NC, n_nodes, d), jnp.float32),
            jax.ShapeDtypeStruct((NC * NS * CNT_PIECE,), jnp.float32),
        ],
        mesh=mesh,
        scratch_types=[
            pltpu.VMEM((IDX_BLOCK, CHUNK), jnp.int32),
            pltpu.VMEM((IDX_BLOCK, CHUNK), jnp.int32),
            pltpu.VMEM((NBUF, CHUNK, d), jnp.float32),
            pltpu.VMEM((128,), jnp.float32),
            pltpu.VMEM((CNT_PIECE,), jnp.float32),
            pltpu.VMEM_SHARED((n_nodes, d), jnp.float32),
            pltpu.VMEM_SHARED((NS * CNT_PIECE,), jnp.float32),
            pltpu.SemaphoreType.DMA,
            pltpu.SemaphoreType.DMA,
            pltpu.SemaphoreType.DMA,
            pltpu.SemaphoreType.DMA,
        ],
    )
    def sc_agg(h_hbm, ei_hbm, sums_hbm, cnts_hbm,
               src_v, dst_v, rows_v, ones_v, cnt_v, acc_sh, cnt_sh,
               sem0, sem1, sem2, sem3):
        sems = (sem0, sem1, sem2, sem3)
        c = lax.axis_index("c")
        s = lax.axis_index("s")
        wid = s * NC + c

        for j in range(128 // 16):
            ones_v[pl.ds(j * 16, 16)] = jnp.ones((16,), jnp.float32)

        z16 = jnp.zeros((16,), jnp.float32)
        zr = (CHUNK // 8) * 8

        def zrow(r, carry):
            for j in range(d // 16):
                rows_v[0, r, pl.ds(j * 16, 16)] = z16
            return carry

        lax.fori_loop(0, zr, zrow, 0)

        def zcnt(k, carry):
            cnt_v[pl.ds(k * 16, 16)] = z16
            return carry

        lax.fori_loop(0, CNT_PIECE // 16, zcnt, 0)

        row0 = s * rows_per_tile
        for k in range(rows_per_tile // zr):
            pltpu.sync_copy(rows_v.at[0, pl.ds(0, zr)],
                            acc_sh.at[pl.ds(row0 + k * zr, zr)])
        tail = rows_per_tile % zr
        if tail:
            pltpu.sync_copy(rows_v.at[0, pl.ds(0, tail)],
                            acc_sh.at[pl.ds(row0 + rows_per_tile - tail, tail)])

        @pl.when(s == NS - 1)
        def _():
            pltpu.sync_copy(rows_v.at[0, pl.ds(0, rem_rows)],
                            acc_sh.at[pl.ds(rem_row0, rem_rows)])

        pltpu.sync_copy(cnt_v, cnt_sh.at[pl.ds(s * CNT_PIECE, CNT_PIECE)])

        plsc.subcore_barrier()

        for blk in range(n_chunks // IDX_BLOCK):
            pltpu.sync_copy(ei_hbm.at[0, wid, pl.ds(blk * IDX_BLOCK, IDX_BLOCK)], src_v)
            pltpu.sync_copy(ei_hbm.at[1, wid, pl.ds(blk * IDX_BLOCK, IDX_BLOCK)], dst_v)

            for b in range(NBUF):
                pltpu.async_copy(h_hbm.at[src_v.at[b]], rows_v.at[b], sems[b])

            def body(g, carry):
                for b in range(NBUF):
                    i = NBUF * g + b
                    pltpu.make_async_copy(h_hbm.at[src_v.at[i]], rows_v.at[b],
                                          sems[b]).wait()
                    pltpu.sync_copy(rows_v.at[b], acc_sh.at[dst_v.at[i]], add=True)
                    pltpu.sync_copy(ones_v.at[pl.ds(0, CHUNK)],
                                    cnt_sh.at[dst_v.at[i]], add=True)

                    @pl.when(i + NBUF < IDX_BLOCK)
                    def _():
                        pltpu.async_copy(h_hbm.at[src_v.at[i + NBUF]],
                                         rows_v.at[b], sems[b])
                return carry

            lax.fori_loop(0, IDX_BLOCK // NBUF, body, 0)

        plsc.subcore_barrier()

        pltpu.sync_copy(acc_sh.at[pl.ds(row0, rows_per_tile)],
                        sums_hbm.at[c, pl.ds(row0, rows_per_tile)])

        @pl.when(s == NS - 1)
        def _():
            pltpu.sync_copy(acc_sh.at[pl.ds(rem_row0, rem_rows)],
                            sums_hbm.at[c, pl.ds(rem_row0, rem_rows)])

        pltpu.sync_copy(cnt_sh.at[pl.ds(s * CNT_PIECE, CNT_PIECE)], cnt_v)
        pltpu.sync_copy(
            cnt_v,
            cnts_hbm.at[pl.ds((c * NS + s) * CNT_PIECE, CNT_PIECE)])

    return sc_agg


def kernel(x, edge_index, weight, bias):
    n_nodes, d_in = x.shape
    d_out = weight.shape[1]
    n_edges = edge_index.shape[1]

    h = pl.pallas_call(
        _transform_body,
        out_shape=jax.ShapeDtypeStruct((n_nodes, d_out), jnp.float32),
    )(x, weight, bias.reshape(1, d_out))

    edges_per_w = n_edges // NW
    n_chunks = edges_per_w // CHUNK
    ei4 = edge_index.reshape(2, NW, n_chunks, CHUNK)
    sums, cnts = _make_sc_agg(n_nodes, n_edges, d_out)(h, ei4)

    out = pl.pallas_call(
        _combine_body,
        out_shape=jax.ShapeDtypeStruct((n_nodes, d_out), jnp.float32),
    )(sums, cnts.reshape(NC, NS * CNT_PIECE)[:, :n_nodes])
    return out


# speedup vs baseline: 1.3957x; 1.3957x over previous
import functools

import jax
import jax.numpy as jnp
from jax import lax
from jax.experimental import pallas as pl
from jax.experimental.pallas import tpu as pltpu
from jax.experimental.pallas import tpu_sc as plsc

MIN_NORM = 1e-7
BALL_EPS = 1e-5

NC = 2
NS = 16
NW = NC * NS

CHUNK = 125
NBUF = 2
IDX_BLOCK = 40
CNT_PIECE = 640


def _transform_body(x_ref, w_ref, b_ref, h_ref):
    w = w_ref[...]
    b = b_ref[...]
    wn = jnp.maximum(jnp.sqrt(jnp.sum(w * w, axis=-1, keepdims=True)), MIN_NORM)
    w_h = jnp.tanh(wn) * w / wn
    bn = jnp.maximum(jnp.sqrt(jnp.sum(b * b, axis=-1, keepdims=True)), MIN_NORM)
    b_h = jnp.tanh(bn) * b / bn
    x = x_ref[...]
    xn = jnp.maximum(jnp.sqrt(jnp.sum(x * x, axis=-1, keepdims=True)), MIN_NORM)
    mx = jnp.dot(x, w_h, preferred_element_type=jnp.float32)
    mxn = jnp.maximum(jnp.sqrt(jnp.sum(mx * mx, axis=-1, keepdims=True)), MIN_NORM)
    xc = jnp.clip(xn, -1.0 + 1e-7, 1.0 - 1e-7)
    art = 0.5 * jnp.log((1.0 + xc) / (1.0 - xc))
    res = jnp.tanh(mxn / xn * art) * mx / mxn
    nz = jnp.max(jnp.abs(mx), axis=-1, keepdims=True) > 0.0
    h = jnp.where(nz, res, 0.0)
    x2 = jnp.sum(h * h, axis=-1, keepdims=True)
    y2 = jnp.sum(b_h * b_h, axis=-1, keepdims=True)
    xy = jnp.sum(h * b_h, axis=-1, keepdims=True)
    num = (1.0 + 2.0 * xy + y2) * h + (1.0 - x2) * b_h
    den = 1.0 + 2.0 * xy + x2 * y2
    h_ref[...] = num / jnp.maximum(den, 1e-15)


def _combine_body(sums_ref, cnts_ref, out_ref):
    s = sums_ref[0] + sums_ref[1]
    c = cnts_ref[0] + cnts_ref[1]
    agg = s / jnp.maximum(c, 1.0)[:, None]
    norm = jnp.maximum(jnp.sqrt(jnp.sum(agg * agg, axis=-1, keepdims=True)), MIN_NORM)
    maxnorm = 1.0 - BALL_EPS
    out_ref[...] = jnp.where(norm > maxnorm, agg / norm * maxnorm, agg)


def _make_sc_agg(n_nodes, n_edges, d):
    edges_per_w = n_edges // NW
    n_chunks = edges_per_w // CHUNK
    rows_per_tile = (n_nodes // NS) // 8 * 8
    rem_row0 = NS * rows_per_tile
    rem_rows = n_nodes - rem_row0
    mesh = plsc.VectorSubcoreMesh(core_axis_name="c", subcore_axis_name="s")

    @functools.partial(
        pl.kernel,
        out_type=[
            jax.ShapeDtypeStruct((NC, n_nodes, d), jnp.float32),
            jax.ShapeDtypeStruct((NC * NS * CNT_PIECE,), jnp.float32),
        ],
        mesh=mesh,
        scratch_types=[
            pltpu.VMEM((IDX_BLOCK, CHUNK), jnp.int32),
            pltpu.VMEM((IDX_BLOCK, CHUNK), jnp.int32),
            pltpu.VMEM((NBUF, CHUNK, d), jnp.float32),
            pltpu.VMEM((128,), jnp.float32),
            pltpu.VMEM((CNT_PIECE,), jnp.float32),
            pltpu.VMEM_SHARED((n_nodes, d), jnp.float32),
            pltpu.VMEM_SHARED((NS * CNT_PIECE,), jnp.float32),
            pltpu.SemaphoreType.DMA,
            pltpu.SemaphoreType.DMA,
            pltpu.SemaphoreType.DMA,
            pltpu.SemaphoreType.DMA,
        ],
    )
    def sc_agg(h_hbm, ei_hbm, sums_hbm, cnts_hbm,
               src_v, dst_v, rows_v, ones_v, cnt_v, acc_sh, cnt_sh,
               sem0, sem1, sem2, sem3):
        sems = (sem0, sem1, sem2, sem3)
        c = lax.axis_index("c")
        s = lax.axis_index("s")
        wid = s * NC + c

        for j in range(128 // 16):
            ones_v[pl.ds(j * 16, 16)] = jnp.ones((16,), jnp.float32)

        z16 = jnp.zeros((16,), jnp.float32)
        zr = (CHUNK // 8) * 8

        def zrow(r, carry):
            for j in range(d // 16):
                rows_v[0, r, pl.ds(j * 16, 16)] = z16
            return carry

        lax.fori_loop(0, zr, zrow, 0)

        def zcnt(k, carry):
            cnt_v[pl.ds(k * 16, 16)] = z16
            return carry

        lax.fori_loop(0, CNT_PIECE // 16, zcnt, 0)

        row0 = s * rows_per_tile
        for k in range(rows_per_tile // zr):
            pltpu.sync_copy(rows_v.at[0, pl.ds(0, zr)],
                            acc_sh.at[pl.ds(row0 + k * zr, zr)])
        tail = rows_per_tile % zr
        if tail:
            pltpu.sync_copy(rows_v.at[0, pl.ds(0, tail)],
                            acc_sh.at[pl.ds(row0 + rows_per_tile - tail, tail)])

        @pl.when(s == NS - 1)
        def _():
            pltpu.sync_copy(rows_v.at[0, pl.ds(0, rem_rows)],
                            acc_sh.at[pl.ds(rem_row0, rem_rows)])

        pltpu.sync_copy(cnt_v, cnt_sh.at[pl.ds(s * CNT_PIECE, CNT_PIECE)])

        plsc.subcore_barrier()

        for blk in range(n_chunks // IDX_BLOCK):
            pltpu.sync_copy(ei_hbm.at[0, wid, pl.ds(blk * IDX_BLOCK, IDX_BLOCK)], src_v)
            pltpu.sync_copy(ei_hbm.at[1, wid, pl.ds(blk * IDX_BLOCK, IDX_BLOCK)], dst_v)

            for b in range(NBUF):
                pltpu.async_copy(h_hbm.at[src_v.at[b]], rows_v.at[b], sems[b])

            def body(g, carry):
                for b in range(NBUF):
                    i = NBUF * g + b
                    pltpu.make_async_copy(h_hbm.at[src_v.at[i]], rows_v.at[b],
                                          sems[b]).wait()
                    pltpu.sync_copy(rows_v.at[b], acc_sh.at[dst_v.at[i]], add=True)
                    pltpu.sync_copy(ones_v.at[pl.ds(0, CHUNK)],
                                    cnt_sh.at[dst_v.at[i]], add=True)

                    @pl.when(i + NBUF < IDX_BLOCK)
                    def _():
                        pltpu.async_copy(h_hbm.at[src_v.at[i + NBUF]],
                                         rows_v.at[b], sems[b])
                return carry

            lax.fori_loop(0, IDX_BLOCK // NBUF, body, 0)

        plsc.subcore_barrier()

        pltpu.sync_copy(acc_sh.at[pl.ds(row0, rows_per_tile)],
                        sums_hbm.at[c, pl.ds(row0, rows_per_tile)])

        @pl.when(s == NS - 1)
        def _():
            pltpu.sync_copy(acc_sh.at[pl.ds(rem_row0, rem_rows)],
                            sums_hbm.at[c, pl.ds(rem_row0, rem_rows)])

        pltpu.sync_copy(cnt_sh.at[pl.ds(s * CNT_PIECE, CNT_PIECE)], cnt_v)
        pltpu.sync_copy(
            cnt_v,
            cnts_hbm.at[pl.ds((c * NS + s) * CNT_PIECE, CNT_PIECE)])

    return sc_agg


def kernel(x, edge_index, weight, bias):
    n_nodes, d_in = x.shape
    d_out = weight.shape[1]
    n_edges = edge_index.shape[1]

    h = pl.pallas_call(
        _transform_body,
        out_shape=jax.ShapeDtypeStruct((n_nodes, d_out), jnp.float32),
    )(x, weight, bias.reshape(1, d_out))

    edges_per_w = n_edges // NW
    n_chunks = edges_per_w // CHUNK
    ei4 = edge_index.reshape(2, NW, n_chunks, CHUNK)
    sums, cnts = _make_sc_agg(n_nodes, n_edges, d_out)(h, ei4)

    out = pl.pallas_call(
        _combine_body,
        out_shape=jax.ShapeDtypeStruct((n_nodes, d_out), jnp.float32),
    )(sums, cnts.reshape(NC, NS * CNT_PIECE)[:, :n_nodes])
    return out
